# Initial kernel scaffold; baseline (speedup 1.0000x reference)
import jax
import jax.numpy as jnp
from jax.experimental import pallas as pl


def kernel(h, surprise, mem, strength):
    raise NotImplementedError("write your pallas kernel here")


# speedup vs baseline: 1.6474x; 1.6474x over previous
import jax
import jax.numpy as jnp
from jax.experimental import pallas as pl
from jax.experimental.pallas import tpu as pltpu

DECAY = 0.98
LR = 0.05
B, T, D = 4, 128, 64
M = 65536
BM = 2048
QT = B * T


def _flash_body(hf_ref, sur_ref, mem_ref, str_ref, out_ref,
                qn_ref, acc_ref, m_ref, l_ref, dmem_ref, sstr_ref):
    j = pl.program_id(0)
    nb = pl.num_programs(0)

    @pl.when(j == 0)
    def _prologue():
        hfv = hf_ref[...]
        qss = jnp.sum(hfv * hfv, axis=1, keepdims=True)
        qn_ref[...] = hfv / jnp.maximum(jnp.sqrt(qss), 1e-12)

        s2 = jnp.mean(sur_ref[...], axis=0, keepdims=True)
        r_io = jax.lax.broadcasted_iota(jnp.int32, (T, T), 0)
        c_io = jax.lax.broadcasted_iota(jnp.int32, (T, T), 1)
        eye = (r_io == c_io).astype(jnp.float32)
        s_bc = jnp.broadcast_to(s2, (T, T))
        s_col = jnp.sum(s_bc * eye, axis=1, keepdims=True)
        gt = (s_col > s2).astype(jnp.int32)
        tie = ((s_col == s2) & (r_io < c_io)).astype(jnp.int32)
        rank = jnp.sum(gt + tie, axis=0, keepdims=True)
        ohot = (jnp.broadcast_to(rank, (T, T)) == r_io).astype(jnp.float32)

        mh = (hfv[0:T] + hfv[T:2 * T] + hfv[2 * T:3 * T] + hfv[3 * T:4 * T]) * 0.25
        mss = jnp.sum(mh * mh, axis=1, keepdims=True)
        mhn = mh / jnp.maximum(jnp.sqrt(mss), 1e-12)
        delta = LR * jax.lax.dot_general(
            ohot, mhn, (((1,), (0,)), ((), ())),
            preferred_element_type=jnp.float32)
        dmem_ref[...] = jnp.concatenate(
            [delta, jnp.zeros((BM - T, D), jnp.float32)], axis=0)

        ss_col = jnp.sum(ohot * s_bc, axis=1, keepdims=True)
        ss_row = jnp.sum(jnp.broadcast_to(ss_col, (T, T)) * eye,
                         axis=0, keepdims=True)
        sstr_ref[...] = jnp.concatenate(
            [ss_row, jnp.zeros((1, BM - T), jnp.float32)], axis=1)

        acc_ref[...] = jnp.zeros((QT, D), jnp.float32)
        m_ref[...] = jnp.full((QT, 1), -1e30, jnp.float32)
        l_ref[...] = jnp.zeros((QT, 1), jnp.float32)

    is0 = jnp.where(j == 0, 1.0, 0.0)
    dec = mem_ref[...] * DECAY + is0 * dmem_ref[...]
    str2 = str_ref[...] * DECAY + is0 * sstr_ref[...]
    nss = jnp.sum(dec * dec, axis=1, keepdims=True)
    mn = dec / jnp.maximum(jnp.sqrt(nss), 1e-12)
    qn = qn_ref[...]
    logits = jax.lax.dot_general(
        qn, mn, (((1,), (1,)), ((), ())),
        preferred_element_type=jnp.float32)
    logits = logits * str2
    bmax = jnp.max(logits, axis=1, keepdims=True)
    m_prev = m_ref[...]
    m_new = jnp.maximum(m_prev, bmax)
    alpha = jnp.exp(m_prev - m_new)
    p = jnp.exp(logits - m_new)
    l_ref[...] = l_ref[...] * alpha + jnp.sum(p, axis=1, keepdims=True)
    acc_ref[...] = acc_ref[...] * alpha + jax.lax.dot_general(
        p, dec, (((1,), (0,)), ((), ())), preferred_element_type=jnp.float32)
    m_ref[...] = m_new

    @pl.when(j == nb - 1)
    def _finalize():
        out_ref[...] = acc_ref[...] / l_ref[...]


def kernel(h, surprise, mem, strength):
    hf = h.reshape(QT, D)
    strr = strength.reshape(1, M)
    out = pl.pallas_call(
        _flash_body,
        grid=(M // BM,),
        in_specs=[
            pl.BlockSpec((QT, D), lambda j: (0, 0)),
            pl.BlockSpec((B, T), lambda j: (0, 0)),
            pl.BlockSpec((BM, D), lambda j: (j, 0)),
            pl.BlockSpec((1, BM), lambda j: (0, j)),
        ],
        out_specs=pl.BlockSpec((QT, D), lambda j: (0, 0)),
        out_shape=jax.ShapeDtypeStruct((QT, D), jnp.float32),
        scratch_shapes=[
            pltpu.VMEM((QT, D), jnp.float32),
            pltpu.VMEM((QT, D), jnp.float32),
            pltpu.VMEM((QT, 1), jnp.float32),
            pltpu.VMEM((QT, 1), jnp.float32),
            pltpu.VMEM((BM, D), jnp.float32),
            pltpu.VMEM((1, BM), jnp.float32),
        ],
        compiler_params=pltpu.CompilerParams(
            dimension_semantics=("arbitrary",)),
    )(hf, surprise, mem, strr)
    return out.reshape(B, T, D)
